# baseline (speedup 1.0000x reference)
import jax
import jax.numpy as jnp
from jax import lax
from jax.experimental import pallas as pl
from jax.experimental.pallas import tpu as pltpu
from jax.experimental.pallas import tpu_sc as plsc

B, S = 4, 2048
HIDDEN = 768
EPS = 1e-12

NC, NS = 2, 16
NW = NC * NS
N_TOK = B * S
N_HALF = 2
HALF = N_TOK // N_HALF
TOK_PER_W = HALF // NW
CHUNK = 32
N_CHUNK = TOK_PER_W // CHUNK

TC_BLK = 1024
S_HALF = S // N_HALF
W_PER_B = NW // B


def _make_sc_body(h):
    def body(ids_hbm, word_hbm, out_hbm,
             idx0, idx1, rows0, rows1, gs0, gs1, os0, os1):
        wid = lax.axis_index("c") * NS + lax.axis_index("s")
        base = ((wid // W_PER_B) * S + h * S_HALF
                + (wid % W_PER_B) * TOK_PER_W)
        idx = (idx0, idx1)
        rows = (rows0, rows1)
        gsem = (gs0, gs1)
        osem = (os0, os1)

        out_copies = [None, None]
        pltpu.sync_copy(ids_hbm.at[pl.ds(base, CHUNK)], idx0)
        cur = pltpu.async_copy(word_hbm.at[idx0], rows0, gs0)
        for c in range(N_CHUNK):
            p = c & 1
            q = p ^ 1
            if c + 1 < N_CHUNK:
                if out_copies[q] is not None:
                    out_copies[q].wait()
                pltpu.sync_copy(
                    ids_hbm.at[pl.ds(base + (c + 1) * CHUNK, CHUNK)], idx[q])
                nxt = pltpu.async_copy(word_hbm.at[idx[q]], rows[q], gsem[q])
            cur.wait()
            out_copies[p] = pltpu.async_copy(
                rows[p],
                out_hbm.at[pl.ds(wid * TOK_PER_W + c * CHUNK, CHUNK)],
                osem[p])
            if c + 1 < N_CHUNK:
                cur = nxt
        for oc in out_copies:
            if oc is not None:
                oc.wait()
    return body


def _sc_gather(ids, word_emb, h):
    mesh = plsc.VectorSubcoreMesh(core_axis_name="c", subcore_axis_name="s",
                                  num_cores=NC, num_subcores=NS)
    f = pl.kernel(
        _make_sc_body(h),
        out_type=jax.ShapeDtypeStruct((HALF, HIDDEN), jnp.float32),
        mesh=mesh,
        compiler_params=pltpu.CompilerParams(needs_layout_passes=False),
        scratch_types=[
            pltpu.VMEM((CHUNK,), jnp.int32),
            pltpu.VMEM((CHUNK,), jnp.int32),
            pltpu.VMEM((CHUNK, HIDDEN), jnp.float32),
            pltpu.VMEM((CHUNK, HIDDEN), jnp.float32),
            pltpu.SemaphoreType.DMA,
            pltpu.SemaphoreType.DMA,
            pltpu.SemaphoreType.DMA,
            pltpu.SemaphoreType.DMA,
        ],
    )
    return f(ids, word_emb)


def _ln_math(x, ttf, type_ref, gam, bet):
    t0 = type_ref[0:1, :]
    dt = type_ref[1:2, :] - t0
    x = x + t0 + ttf * dt
    mean = jnp.mean(x, axis=-1, keepdims=True)
    cent = x - mean
    var = jnp.mean(cent * cent, axis=-1, keepdims=True)
    return cent * lax.rsqrt(var + EPS) * gam + bet


def _tc_body_first(gath_ref, pos_ref, ttf_ref, type_ref, gam_ref, bet_ref,
                   out_ref):
    out_ref[...] = _ln_math(gath_ref[...] + pos_ref[...], ttf_ref[...],
                            type_ref, gam_ref[...], bet_ref[...])


def _tc_body_second(prev_ref, gath_ref, pos_ref, ttf_ref, type_ref, gam_ref,
                    bet_ref, out_ref):
    del prev_ref
    out_ref[...] = _ln_math(gath_ref[...] + pos_ref[...], ttf_ref[...],
                            type_ref, gam_ref[...], bet_ref[...])


def _tc_ln(gathered, ttf2d, pos_emb, type_emb, gamma, beta, h, prev):
    common_specs = [
        pl.BlockSpec((TC_BLK, HIDDEN), lambda b: (b, 0)),
        pl.BlockSpec((TC_BLK, HIDDEN), lambda b: (h, 0)),
        pl.BlockSpec((TC_BLK, 1), lambda b: (b * N_HALF + h, 0)),
        pl.BlockSpec((2, HIDDEN), lambda b: (0, 0)),
        pl.BlockSpec((1, HIDDEN), lambda b: (0, 0)),
        pl.BlockSpec((1, HIDDEN), lambda b: (0, 0)),
    ]
    out_spec = pl.BlockSpec((TC_BLK, HIDDEN), lambda b: (b * N_HALF + h, 0))
    out_shape = jax.ShapeDtypeStruct((N_TOK, HIDDEN), jnp.float32)
    if prev is None:
        return pl.pallas_call(
            _tc_body_first, grid=(B,),
            in_specs=common_specs, out_specs=out_spec, out_shape=out_shape,
        )(gathered, pos_emb, ttf2d, type_emb, gamma, beta)
    return pl.pallas_call(
        _tc_body_second, grid=(B,),
        in_specs=[pl.BlockSpec(memory_space=pl.ANY)]
        + common_specs,
        out_specs=out_spec, out_shape=out_shape,
        input_output_aliases={0: 0},
    )(prev, gathered, pos_emb, ttf2d, type_emb, gamma, beta)


@jax.jit
def _run(ids, ttf2d, word_emb, pos_emb, type_emb, gamma2d, beta2d):
    out = None
    gath = [_sc_gather(ids, word_emb, h) for h in range(N_HALF)]
    for h in range(N_HALF):
        out = _tc_ln(gath[h], ttf2d, pos_emb, type_emb, gamma2d, beta2d,
                     h, out)
    return out


def kernel(input_ids, token_type_ids, word_emb, pos_emb, type_emb, gamma, beta):
    ids = input_ids.reshape(-1).astype(jnp.int32)
    ttf2d = token_type_ids.reshape(-1, 1).astype(jnp.float32)
    out = _run(ids, ttf2d, word_emb, pos_emb, type_emb,
               gamma.reshape(1, HIDDEN), beta.reshape(1, HIDDEN))
    return out.reshape(B, S, HIDDEN)


# speedup vs baseline: 1.0067x; 1.0067x over previous
import jax
import jax.numpy as jnp
from jax import lax
from jax.experimental import pallas as pl
from jax.experimental.pallas import tpu as pltpu
from jax.experimental.pallas import tpu_sc as plsc

B, S = 4, 2048
HIDDEN = 768
EPS = 1e-12

NC, NS = 2, 16
NW = NC * NS
N_TOK = B * S
N_HALF = 2
HALF = N_TOK // N_HALF
TOK_PER_W = HALF // NW
CHUNK = 32
N_CHUNK = TOK_PER_W // CHUNK

TC_BLK = 1024
S_HALF = S // N_HALF
W_PER_B = NW // B


def _make_sc_body(h):
    def body(ids_hbm, word_hbm, out_hbm, *refs):
        idx = refs[:N_CHUNK]
        rows = refs[N_CHUNK:2 * N_CHUNK]
        gsem = refs[2 * N_CHUNK:3 * N_CHUNK]
        osem = refs[3 * N_CHUNK:4 * N_CHUNK]
        wid = lax.axis_index("c") * NS + lax.axis_index("s")
        base = ((wid // W_PER_B) * S + h * S_HALF
                + (wid % W_PER_B) * TOK_PER_W)

        gathers = []
        for c in range(N_CHUNK):
            pltpu.sync_copy(ids_hbm.at[pl.ds(base + c * CHUNK, CHUNK)],
                            idx[c])
            gathers.append(
                pltpu.async_copy(word_hbm.at[idx[c]], rows[c], gsem[c]))
        outs = []
        for c in range(N_CHUNK):
            gathers[c].wait()
            outs.append(pltpu.async_copy(
                rows[c],
                out_hbm.at[pl.ds(wid * TOK_PER_W + c * CHUNK, CHUNK)],
                osem[c]))
        for oc in outs:
            oc.wait()
    return body


def _sc_gather(ids, word_emb, h):
    mesh = plsc.VectorSubcoreMesh(core_axis_name="c", subcore_axis_name="s",
                                  num_cores=NC, num_subcores=NS)
    f = pl.kernel(
        _make_sc_body(h),
        out_type=jax.ShapeDtypeStruct((HALF, HIDDEN), jnp.float32),
        mesh=mesh,
        compiler_params=pltpu.CompilerParams(needs_layout_passes=False),
        scratch_types=(
            [pltpu.VMEM((CHUNK,), jnp.int32) for _ in range(N_CHUNK)]
            + [pltpu.VMEM((CHUNK, HIDDEN), jnp.float32)
               for _ in range(N_CHUNK)]
            + [pltpu.SemaphoreType.DMA for _ in range(2 * N_CHUNK)]
        ),
    )
    return f(ids, word_emb)


def _ln_math(x, ttf, type_ref, gam, bet):
    t0 = type_ref[0:1, :]
    dt = type_ref[1:2, :] - t0
    x = x + t0 + ttf * dt
    mean = jnp.mean(x, axis=-1, keepdims=True)
    cent = x - mean
    var = jnp.mean(cent * cent, axis=-1, keepdims=True)
    return cent * lax.rsqrt(var + EPS) * gam + bet


def _tc_body_first(gath_ref, pos_ref, ttf_ref, type_ref, gam_ref, bet_ref,
                   out_ref):
    out_ref[...] = _ln_math(gath_ref[...] + pos_ref[...], ttf_ref[...],
                            type_ref, gam_ref[...], bet_ref[...])


def _tc_body_second(prev_ref, gath_ref, pos_ref, ttf_ref, type_ref, gam_ref,
                    bet_ref, out_ref):
    del prev_ref
    out_ref[...] = _ln_math(gath_ref[...] + pos_ref[...], ttf_ref[...],
                            type_ref, gam_ref[...], bet_ref[...])


def _tc_ln(gathered, ttf2d, pos_emb, type_emb, gamma, beta, h, prev):
    common_specs = [
        pl.BlockSpec((TC_BLK, HIDDEN), lambda b: (b, 0)),
        pl.BlockSpec((TC_BLK, HIDDEN), lambda b: (h, 0)),
        pl.BlockSpec((TC_BLK, 1), lambda b: (b * N_HALF + h, 0)),
        pl.BlockSpec((2, HIDDEN), lambda b: (0, 0)),
        pl.BlockSpec((1, HIDDEN), lambda b: (0, 0)),
        pl.BlockSpec((1, HIDDEN), lambda b: (0, 0)),
    ]
    out_spec = pl.BlockSpec((TC_BLK, HIDDEN), lambda b: (b * N_HALF + h, 0))
    out_shape = jax.ShapeDtypeStruct((N_TOK, HIDDEN), jnp.float32)
    if prev is None:
        return pl.pallas_call(
            _tc_body_first, grid=(B,),
            in_specs=common_specs, out_specs=out_spec, out_shape=out_shape,
        )(gathered, pos_emb, ttf2d, type_emb, gamma, beta)
    return pl.pallas_call(
        _tc_body_second, grid=(B,),
        in_specs=[pl.BlockSpec(memory_space=pl.ANY)]
        + common_specs,
        out_specs=out_spec, out_shape=out_shape,
        input_output_aliases={0: 0},
    )(prev, gathered, pos_emb, ttf2d, type_emb, gamma, beta)


@jax.jit
def _run(ids, ttf2d, word_emb, pos_emb, type_emb, gamma2d, beta2d):
    out = None
    gath = [_sc_gather(ids, word_emb, h) for h in range(N_HALF)]
    for h in range(N_HALF):
        out = _tc_ln(gath[h], ttf2d, pos_emb, type_emb, gamma2d, beta2d,
                     h, out)
    return out


def kernel(input_ids, token_type_ids, word_emb, pos_emb, type_emb, gamma, beta):
    ids = input_ids.reshape(-1).astype(jnp.int32)
    ttf2d = token_type_ids.reshape(-1, 1).astype(jnp.float32)
    out = _run(ids, ttf2d, word_emb, pos_emb, type_emb,
               gamma.reshape(1, HIDDEN), beta.reshape(1, HIDDEN))
    return out.reshape(B, S, HIDDEN)


# speedup vs baseline: 1.0248x; 1.0180x over previous
import jax
import jax.numpy as jnp
from jax import lax
from jax.experimental import pallas as pl
from jax.experimental.pallas import tpu as pltpu
from jax.experimental.pallas import tpu_sc as plsc

B, S = 4, 2048
HIDDEN = 768
EPS = 1e-12

NC, NS = 2, 16
NW = NC * NS
N_TOK = B * S
N_HALF = 2
HALF = N_TOK // N_HALF
TOK_PER_W = HALF // NW
CHUNK = 64
N_CHUNK = TOK_PER_W // CHUNK

TC_BLK = 1024
S_HALF = S // N_HALF
W_PER_B = NW // B


def _make_sc_body(h):
    def body(ids_hbm, word_hbm, out_hbm, *refs):
        idx = refs[:N_CHUNK]
        rows = refs[N_CHUNK:2 * N_CHUNK]
        gsem = refs[2 * N_CHUNK:3 * N_CHUNK]
        osem = refs[3 * N_CHUNK:4 * N_CHUNK]
        wid = lax.axis_index("c") * NS + lax.axis_index("s")
        base = ((wid // W_PER_B) * S + h * S_HALF
                + (wid % W_PER_B) * TOK_PER_W)

        gathers = []
        for c in range(N_CHUNK):
            pltpu.sync_copy(ids_hbm.at[pl.ds(base + c * CHUNK, CHUNK)],
                            idx[c])
            gathers.append(
                pltpu.async_copy(word_hbm.at[idx[c]], rows[c], gsem[c]))
        outs = []
        for c in range(N_CHUNK):
            gathers[c].wait()
            outs.append(pltpu.async_copy(
                rows[c],
                out_hbm.at[pl.ds(wid * TOK_PER_W + c * CHUNK, CHUNK)],
                osem[c]))
        for oc in outs:
            oc.wait()
    return body


def _sc_gather(ids, word_emb, h):
    mesh = plsc.VectorSubcoreMesh(core_axis_name="c", subcore_axis_name="s",
                                  num_cores=NC, num_subcores=NS)
    f = pl.kernel(
        _make_sc_body(h),
        out_type=jax.ShapeDtypeStruct((HALF, HIDDEN), jnp.float32),
        mesh=mesh,
        compiler_params=pltpu.CompilerParams(needs_layout_passes=False),
        scratch_types=(
            [pltpu.VMEM((CHUNK,), jnp.int32) for _ in range(N_CHUNK)]
            + [pltpu.VMEM((CHUNK, HIDDEN), jnp.float32)
               for _ in range(N_CHUNK)]
            + [pltpu.SemaphoreType.DMA for _ in range(2 * N_CHUNK)]
        ),
    )
    return f(ids, word_emb)


def _ln_math(x, ttf, type_ref, gam, bet):
    t0 = type_ref[0:1, :]
    dt = type_ref[1:2, :] - t0
    x = x + t0 + ttf * dt
    mean = jnp.mean(x, axis=-1, keepdims=True)
    cent = x - mean
    var = jnp.mean(cent * cent, axis=-1, keepdims=True)
    return cent * lax.rsqrt(var + EPS) * gam + bet


def _tc_body_first(gath_ref, pos_ref, ttf_ref, type_ref, gam_ref, bet_ref,
                   out_ref):
    out_ref[...] = _ln_math(gath_ref[...] + pos_ref[...], ttf_ref[...],
                            type_ref, gam_ref[...], bet_ref[...])


def _tc_body_second(prev_ref, gath_ref, pos_ref, ttf_ref, type_ref, gam_ref,
                    bet_ref, out_ref):
    del prev_ref
    out_ref[...] = _ln_math(gath_ref[...] + pos_ref[...], ttf_ref[...],
                            type_ref, gam_ref[...], bet_ref[...])


def _tc_ln(gathered, ttf2d, pos_emb, type_emb, gamma, beta, h, prev):
    common_specs = [
        pl.BlockSpec((TC_BLK, HIDDEN), lambda b: (b, 0)),
        pl.BlockSpec((TC_BLK, HIDDEN), lambda b: (h, 0)),
        pl.BlockSpec((TC_BLK, 1), lambda b: (b * N_HALF + h, 0)),
        pl.BlockSpec((2, HIDDEN), lambda b: (0, 0)),
        pl.BlockSpec((1, HIDDEN), lambda b: (0, 0)),
        pl.BlockSpec((1, HIDDEN), lambda b: (0, 0)),
    ]
    out_spec = pl.BlockSpec((TC_BLK, HIDDEN), lambda b: (b * N_HALF + h, 0))
    out_shape = jax.ShapeDtypeStruct((N_TOK, HIDDEN), jnp.float32)
    if prev is None:
        return pl.pallas_call(
            _tc_body_first, grid=(B,),
            in_specs=common_specs, out_specs=out_spec, out_shape=out_shape,
        )(gathered, pos_emb, ttf2d, type_emb, gamma, beta)
    return pl.pallas_call(
        _tc_body_second, grid=(B,),
        in_specs=[pl.BlockSpec(memory_space=pl.ANY)]
        + common_specs,
        out_specs=out_spec, out_shape=out_shape,
        input_output_aliases={0: 0},
    )(prev, gathered, pos_emb, ttf2d, type_emb, gamma, beta)


@jax.jit
def _run(ids, ttf2d, word_emb, pos_emb, type_emb, gamma2d, beta2d):
    out = None
    gath = [_sc_gather(ids, word_emb, h) for h in range(N_HALF)]
    for h in range(N_HALF):
        out = _tc_ln(gath[h], ttf2d, pos_emb, type_emb, gamma2d, beta2d,
                     h, out)
    return out


def kernel(input_ids, token_type_ids, word_emb, pos_emb, type_emb, gamma, beta):
    ids = input_ids.reshape(-1).astype(jnp.int32)
    ttf2d = token_type_ids.reshape(-1, 1).astype(jnp.float32)
    out = _run(ids, ttf2d, word_emb, pos_emb, type_emb,
               gamma.reshape(1, HIDDEN), beta.reshape(1, HIDDEN))
    return out.reshape(B, S, HIDDEN)
